# Initial kernel scaffold; baseline (speedup 1.0000x reference)
import jax
import jax.numpy as jnp
from jax.experimental import pallas as pl


def kernel(pred, target):
    raise NotImplementedError("write your pallas kernel here")


# speedup vs baseline: 1.1397x; 1.1397x over previous
import functools

import jax
import jax.numpy as jnp
from jax import lax
from jax.experimental import pallas as pl
from jax.experimental.pallas import tpu as pltpu
from jax.experimental.pallas import tpu_sc as plsc

_TOPK = (1, 5)
_THR = 0.0


def _sc_gather_slivers(pred128, t32, num_rows, num_cols):
    info = plsc.get_sparse_core_info()
    nw = info.num_cores * info.num_subcores
    b_per_w = num_rows // nw
    assert num_rows % nw == 0 and b_per_w % 16 == 0

    mesh = plsc.VectorSubcoreMesh(core_axis_name="c", subcore_axis_name="s")

    @functools.partial(
        pl.kernel,
        mesh=mesh,
        out_type=jax.ShapeDtypeStruct((num_rows, 128), jnp.float32),
        scratch_types=[
            pltpu.VMEM((b_per_w,), jnp.int32),
            pltpu.VMEM((b_per_w,), jnp.int32),
            pltpu.VMEM((b_per_w, 128), jnp.float32),
            pltpu.SemaphoreType.DMA,
        ],
    )
    def gather_kernel(pred_hbm, t_hbm, out_hbm, t_v, idx_v, rows_v, sem):
        wid = lax.axis_index("s") * info.num_cores + lax.axis_index("c")
        base = wid * b_per_w
        pltpu.sync_copy(t_hbm.at[pl.ds(base, b_per_w)], t_v)
        for k in range(b_per_w // 16):
            t16 = t_v[pl.ds(k * 16, 16)]
            rows16 = base + k * 16 + lax.iota(jnp.int32, 16)
            flat16 = rows16 * num_cols + t16
            idx_v[pl.ds(k * 16, 16)] = flat16 >> 7
        pltpu.async_copy(pred_hbm.at[idx_v], rows_v, sem).wait()
        pltpu.sync_copy(rows_v, out_hbm.at[pl.ds(base, b_per_w)])

    return gather_kernel(pred128, t32)


def _tc_rank_scan(pred, t2d, slivers, num_rows, num_cols, cb):
    nb = (num_cols + cb - 1) // cb

    def body(pred_ref, t_ref, sliv_ref, out_ref, acc_ref, s_ref):
        c = pl.program_id(0)
        t = t_ref[...]

        @pl.when(c == 0)
        def _init():
            acc_ref[...] = jnp.zeros_like(acc_ref)
            sliv = sliv_ref[...]
            row = lax.broadcasted_iota(jnp.int32, (num_rows, 1), 0)
            off = (row * num_cols + t) & 127
            lane128 = lax.broadcasted_iota(jnp.int32, (num_rows, 128), 1)
            picked = jnp.where(lane128 == off, sliv, 0.0)
            s_ref[...] = jnp.sum(picked, axis=1, keepdims=True)

        v = pred_ref[...]
        s = s_ref[...]
        col0 = c * cb
        rel = lax.broadcasted_iota(jnp.int32, (num_rows, cb), 1)
        gt = (v > s) & (rel < (num_cols - col0))
        eqb = (v == s) & (rel < (t - col0))
        cnt = (gt | eqb).astype(jnp.int32)
        part = cnt[:, 0:128]
        for k in range(1, cb // 128):
            part = part + cnt[:, k * 128:(k + 1) * 128]
        acc_ref[...] += part

        @pl.when(c == nb - 1)
        def _fin():
            rank = jnp.sum(acc_ref[...], axis=1, keepdims=True)
            ok = s_ref[...] > _THR
            t1 = jnp.sum(((rank < _TOPK[0]) & ok).astype(jnp.float32))
            t5 = jnp.sum(((rank < _TOPK[1]) & ok).astype(jnp.float32))
            lanes = lax.broadcasted_iota(jnp.int32, (1, 2), 1)
            out_ref[...] = jnp.where(lanes == 0, t1, t5) * (100.0 / num_rows)

    return pl.pallas_call(
        body,
        grid=(nb,),
        in_specs=[
            pl.BlockSpec((num_rows, cb), lambda c: (0, c)),
            pl.BlockSpec((num_rows, 1), lambda c: (0, 0)),
            pl.BlockSpec((num_rows, 128), lambda c: (0, 0)),
        ],
        out_specs=pl.BlockSpec((1, 2), lambda c: (0, 0)),
        out_shape=jax.ShapeDtypeStruct((1, 2), jnp.float32),
        scratch_shapes=[pltpu.VMEM((num_rows, 128), jnp.int32),
                        pltpu.VMEM((num_rows, 1), jnp.float32)],
        compiler_params=pltpu.CompilerParams(
            dimension_semantics=("arbitrary",)),
    )(pred, t2d, slivers)


def kernel(pred, target):
    num_rows, num_cols = pred.shape
    t32 = target.astype(jnp.int32)
    pred128 = pred.reshape(num_rows * num_cols // 128, 128)
    slivers = _sc_gather_slivers(pred128, t32, num_rows, num_cols)
    out = _tc_rank_scan(pred, t32.reshape(num_rows, 1), slivers,
                        num_rows, num_cols, cb=1024)
    return out.reshape(2)


# speedup vs baseline: 1.1686x; 1.0254x over previous
import functools

import jax
import jax.numpy as jnp
from jax import lax
from jax.experimental import pallas as pl
from jax.experimental.pallas import tpu as pltpu
from jax.experimental.pallas import tpu_sc as plsc

_TOPK = (1, 5)
_THR = 0.0


def _sc_gather_slivers(pred128, t32, num_rows, num_cols):
    info = plsc.get_sparse_core_info()
    nw = info.num_cores * info.num_subcores
    b_per_w = num_rows // nw
    assert num_rows % nw == 0 and b_per_w % 16 == 0

    mesh = plsc.VectorSubcoreMesh(core_axis_name="c", subcore_axis_name="s")

    @functools.partial(
        pl.kernel,
        mesh=mesh,
        out_type=jax.ShapeDtypeStruct((num_rows, 128), jnp.float32),
        scratch_types=[
            pltpu.VMEM((b_per_w,), jnp.int32),
            pltpu.VMEM((b_per_w,), jnp.int32),
            pltpu.VMEM((b_per_w, 128), jnp.float32),
            pltpu.SemaphoreType.DMA,
        ],
    )
    def gather_kernel(pred_hbm, t_hbm, out_hbm, t_v, idx_v, rows_v, sem):
        wid = lax.axis_index("s") * info.num_cores + lax.axis_index("c")
        base = wid * b_per_w
        pltpu.sync_copy(t_hbm.at[pl.ds(base, b_per_w)], t_v)
        for k in range(b_per_w // 16):
            t16 = t_v[pl.ds(k * 16, 16)]
            rows16 = base + k * 16 + lax.iota(jnp.int32, 16)
            flat16 = rows16 * num_cols + t16
            idx_v[pl.ds(k * 16, 16)] = flat16 >> 7
        pltpu.async_copy(pred_hbm.at[idx_v], rows_v, sem).wait()
        pltpu.sync_copy(rows_v, out_hbm.at[pl.ds(base, b_per_w)])

    return gather_kernel(pred128, t32)


def _tc_rank_scan(pred, t2d, slivers, num_rows, num_cols, cb):
    nb = (num_cols + cb - 1) // cb

    def body(pred_ref, t_ref, sliv_ref, out_ref, acc_ref, s_ref):
        c = pl.program_id(0)
        t = t_ref[...]

        @pl.when(c == 0)
        def _init():
            acc_ref[...] = jnp.zeros_like(acc_ref)
            sliv = sliv_ref[...]
            row = lax.broadcasted_iota(jnp.int32, (num_rows, 1), 0)
            off = (row * num_cols + t) & 127
            lane128 = lax.broadcasted_iota(jnp.int32, (num_rows, 128), 1)
            picked = jnp.where(lane128 == off, sliv, 0.0)
            s_ref[...] = jnp.sum(picked, axis=1, keepdims=True)

        v = pred_ref[...]
        s = s_ref[...]
        col0 = c * cb
        rel = lax.broadcasted_iota(jnp.int32, (num_rows, cb), 1)
        gt = (v > s) & (rel < (num_cols - col0))
        eqb = (v == s) & (rel < (t - col0))
        cnt = (gt | eqb).astype(jnp.int32)
        part = cnt[:, 0:128]
        for k in range(1, cb // 128):
            part = part + cnt[:, k * 128:(k + 1) * 128]
        acc_ref[...] += part

        @pl.when(c == nb - 1)
        def _fin():
            rank = jnp.sum(acc_ref[...], axis=1, keepdims=True)
            ok = s_ref[...] > _THR
            t1 = jnp.sum(((rank < _TOPK[0]) & ok).astype(jnp.float32))
            t5 = jnp.sum(((rank < _TOPK[1]) & ok).astype(jnp.float32))
            lanes = lax.broadcasted_iota(jnp.int32, (1, 2), 1)
            out_ref[...] = jnp.where(lanes == 0, t1, t5) * (100.0 / num_rows)

    return pl.pallas_call(
        body,
        grid=(nb,),
        in_specs=[
            pl.BlockSpec((num_rows, cb), lambda c: (0, c)),
            pl.BlockSpec((num_rows, 1), lambda c: (0, 0)),
            pl.BlockSpec((num_rows, 128), lambda c: (0, 0)),
        ],
        out_specs=pl.BlockSpec((1, 2), lambda c: (0, 0)),
        out_shape=jax.ShapeDtypeStruct((1, 2), jnp.float32),
        scratch_shapes=[pltpu.VMEM((num_rows, 128), jnp.int32),
                        pltpu.VMEM((num_rows, 1), jnp.float32)],
        compiler_params=pltpu.CompilerParams(
            dimension_semantics=("arbitrary",)),
    )(pred, t2d, slivers)


def kernel(pred, target):
    num_rows, num_cols = pred.shape
    t32 = target.astype(jnp.int32)
    pred128 = pred.reshape(num_rows * num_cols // 128, 128)
    slivers = _sc_gather_slivers(pred128, t32, num_rows, num_cols)
    out = _tc_rank_scan(pred, t32.reshape(num_rows, 1), slivers,
                        num_rows, num_cols, cb=2048)
    return out.reshape(2)
